# Initial kernel scaffold; baseline (speedup 1.0000x reference)
import jax
import jax.numpy as jnp
from jax.experimental import pallas as pl


def kernel(user, rsid, user_table, rsid_table):
    raise NotImplementedError("write your pallas kernel here")


# speedup vs baseline: 1.0584x; 1.0584x over previous
import functools

import jax
import jax.numpy as jnp
from jax import lax
from jax.experimental import pallas as pl
from jax.experimental.pallas import tpu as pltpu
from jax.experimental.pallas import tpu_sc as plsc

F = 128
B = 16384
NW = 32
BPW = B // NW
CH = 128
NCH = BPW // CH
L = 16


def _body(user_hbm, rsid_hbm, ut_hbm, rt_hbm, out_hbm,
          u_idx, r_idx, u_rows, r_rows, out_v, sem_u, sem_r):
    wid = lax.axis_index("s") * 2 + lax.axis_index("c")
    base = wid * BPW

    for c in range(NCH):
        pltpu.sync_copy(user_hbm.at[pl.ds(base + c * CH, CH)], u_idx.at[c])
        pltpu.sync_copy(rsid_hbm.at[pl.ds(base + c * CH, CH)], r_idx.at[c])

    def start(c):
        buf = c % 2
        cu = pltpu.async_copy(ut_hbm.at[u_idx.at[c]], u_rows.at[buf], sem_u)
        cr = pltpu.async_copy(rt_hbm.at[r_idx.at[c]], r_rows.at[buf], sem_r)
        return cu, cr

    pending = start(0)

    lanes = lax.iota(jnp.int32, L)
    perms = [lanes ^ k for k in (8, 4, 2, 1)]
    dnums = lax.GatherDimensionNumbers(
        offset_dims=(), collapsed_slice_dims=(0,), start_index_map=(0,))

    def lane_sum(x):
        for p in perms:
            x = x + lax.gather(
                x, p[:, None], dimension_numbers=dnums, slice_sizes=(1,),
                mode=lax.GatherScatterMode.PROMISE_IN_BOUNDS)
        return x

    for c in range(NCH):
        buf = c % 2
        cu, cr = pending
        cu.wait()
        cr.wait()
        if c + 1 < NCH:
            pending = start(c + 1)

        def group_body(g, _, buf=buf, c=c):
            rowbase = g * L
            outv = jnp.zeros((L,), jnp.float32)
            for i in range(L):
                row = rowbase + i
                acc = (u_rows[buf, row, pl.ds(0, L)]
                       * r_rows[buf, row, pl.ds(0, L)])
                for fs in range(1, F // L):
                    acc += (u_rows[buf, row, pl.ds(fs * L, L)]
                            * r_rows[buf, row, pl.ds(fs * L, L)])
                s = lane_sum(acc)
                outv = jnp.where(lanes == i, s, outv)
            out_v[pl.ds(c * CH + rowbase, L)] = outv
            return 0

        lax.fori_loop(0, CH // L, group_body, 0)

    pltpu.sync_copy(out_v, out_hbm.at[pl.ds(base, BPW)])


@jax.jit
def _run(user, rsid, user_table, rsid_table):
    mesh = plsc.VectorSubcoreMesh(core_axis_name="c", subcore_axis_name="s")
    k = functools.partial(
        pl.kernel,
        out_type=jax.ShapeDtypeStruct((B,), jnp.float32),
        mesh=mesh,
        scratch_types=[
            pltpu.VMEM((NCH, CH), jnp.int32),
            pltpu.VMEM((NCH, CH), jnp.int32),
            pltpu.VMEM((2, CH, F), jnp.float32),
            pltpu.VMEM((2, CH, F), jnp.float32),
            pltpu.VMEM((BPW,), jnp.float32),
            pltpu.SemaphoreType.DMA,
            pltpu.SemaphoreType.DMA,
        ],
    )(_body)
    return k(user, rsid, user_table, rsid_table)


def kernel(user, rsid, user_table, rsid_table):
    return _run(user, rsid, user_table, rsid_table)


# speedup vs baseline: 1.2847x; 1.2139x over previous
import functools

import jax
import jax.numpy as jnp
from jax import lax
from jax.experimental import pallas as pl
from jax.experimental.pallas import tpu as pltpu
from jax.experimental.pallas import tpu_sc as plsc

F = 128
B = 16384
NW = 32
BPW = B // NW
CH = 128
NCH = BPW // CH
L = 16
RG = 16


def _body(user_hbm, rsid_hbm, ut_hbm, rt_hbm, out_hbm,
          u_idx, r_idx, u_rows, r_rows, out_v, sem_u, sem_r, sem_i):
    wid = lax.axis_index("s") * 2 + lax.axis_index("c")
    base = wid * BPW

    pltpu.async_copy(user_hbm.at[pl.ds(base, CH)], u_idx.at[0], sem_u)
    pltpu.async_copy(rsid_hbm.at[pl.ds(base, CH)], r_idx.at[0], sem_r)
    rest = []
    for c in range(1, NCH):
        rest.append(pltpu.async_copy(
            user_hbm.at[pl.ds(base + c * CH, CH)], u_idx.at[c], sem_i))
        rest.append(pltpu.async_copy(
            rsid_hbm.at[pl.ds(base + c * CH, CH)], r_idx.at[c], sem_i))
    pltpu.make_async_copy(user_hbm.at[pl.ds(base, CH)], u_idx.at[0],
                          sem_u).wait()
    pltpu.make_async_copy(rsid_hbm.at[pl.ds(base, CH)], r_idx.at[0],
                          sem_r).wait()

    def start(c):
        buf = c % 2
        cu = pltpu.async_copy(ut_hbm.at[u_idx.at[c]], u_rows.at[buf], sem_u)
        cr = pltpu.async_copy(rt_hbm.at[r_idx.at[c]], r_rows.at[buf], sem_r)
        return cu, cr

    pending = start(0)
    for h in rest:
        h.wait()

    lanes = lax.iota(jnp.int32, L)
    perms = {k: lanes ^ k for k in (8, 4, 2, 1)}
    masks = {k: (lanes & k) == 0 for k in (8, 4, 2, 1)}
    dnums = lax.GatherDimensionNumbers(
        offset_dims=(), collapsed_slice_dims=(0,), start_index_map=(0,))

    def perm(x, k):
        return lax.gather(
            x, perms[k][:, None], dimension_numbers=dnums, slice_sizes=(1,),
            mode=lax.GatherScatterMode.PROMISE_IN_BOUNDS)

    def combine(a, b, k):
        return jnp.where(masks[k], a + perm(a, k), b + perm(b, k))

    for c in range(NCH):
        buf = c % 2
        cu, cr = pending
        cu.wait()
        cr.wait()
        if c + 1 < NCH:
            pending = start(c + 1)

        @plsc.parallel_loop(0, CH, step=RG)
        def _rows(r0, buf=buf, c=c):
            accs = []
            for i in range(RG):
                row = r0 + i
                prods = [u_rows[buf, row, pl.ds(fs * L, L)]
                         * r_rows[buf, row, pl.ds(fs * L, L)]
                         for fs in range(F // L)]
                while len(prods) > 1:
                    prods = [a + b for a, b in zip(prods[::2], prods[1::2])]
                accs.append(prods[0])
            k = RG
            while len(accs) > 1:
                k //= 2
                half = len(accs) // 2
                accs = [combine(accs[j], accs[j + half], k * (L // RG))
                        for j in range(half)]
            out_v[pl.ds(c * CH + r0, L)] = accs[0]

    pltpu.sync_copy(out_v, out_hbm.at[pl.ds(base, BPW)])


@jax.jit
def _run(user, rsid, user_table, rsid_table):
    mesh = plsc.VectorSubcoreMesh(core_axis_name="c", subcore_axis_name="s")
    k = functools.partial(
        pl.kernel,
        out_type=jax.ShapeDtypeStruct((B,), jnp.float32),
        mesh=mesh,
        scratch_types=[
            pltpu.VMEM((NCH, CH), jnp.int32),
            pltpu.VMEM((NCH, CH), jnp.int32),
            pltpu.VMEM((2, CH, F), jnp.float32),
            pltpu.VMEM((2, CH, F), jnp.float32),
            pltpu.VMEM((BPW,), jnp.float32),
            pltpu.SemaphoreType.DMA,
            pltpu.SemaphoreType.DMA,
            pltpu.SemaphoreType.DMA,
        ],
    )(_body)
    return k(user, rsid, user_table, rsid_table)


def kernel(user, rsid, user_table, rsid_table):
    return _run(user, rsid, user_table, rsid_table)


# speedup vs baseline: 1.5147x; 1.1790x over previous
import functools

import jax
import jax.numpy as jnp
from jax import lax
from jax.experimental import pallas as pl
from jax.experimental.pallas import tpu as pltpu
from jax.experimental.pallas import tpu_sc as plsc

F = 128
B = 16384
NW = 32
BPW = B // NW
CH = 128
NCH = BPW // CH
L = 16
RG = 16


def _body(user_hbm, rsid_hbm, ut_hbm, rt_hbm, out_hbm,
          u_idx, r_idx, u_rows, r_rows, out_v, sem_u, sem_r, sem_i):
    wid = lax.axis_index("s") * 2 + lax.axis_index("c")
    base = wid * BPW

    pltpu.async_copy(user_hbm.at[pl.ds(base, CH)], u_idx.at[0], sem_u)
    pltpu.async_copy(rsid_hbm.at[pl.ds(base, CH)], r_idx.at[0], sem_r)
    rest = []
    for c in range(1, NCH):
        rest.append(pltpu.async_copy(
            user_hbm.at[pl.ds(base + c * CH, CH)], u_idx.at[c], sem_i))
        rest.append(pltpu.async_copy(
            rsid_hbm.at[pl.ds(base + c * CH, CH)], r_idx.at[c], sem_i))
    pltpu.make_async_copy(user_hbm.at[pl.ds(base, CH)], u_idx.at[0],
                          sem_u).wait()
    pltpu.make_async_copy(rsid_hbm.at[pl.ds(base, CH)], r_idx.at[0],
                          sem_r).wait()

    def start(c):
        buf = c % 2
        cu = pltpu.async_copy(ut_hbm.at[u_idx.at[c]], u_rows.at[buf], sem_u)
        cr = pltpu.async_copy(rt_hbm.at[r_idx.at[c]], r_rows.at[buf], sem_r)
        return cu, cr

    pending = start(0)
    for h in rest:
        h.wait()

    lanes = lax.iota(jnp.int32, L)
    perms = {k: lanes ^ k for k in (8, 4, 2, 1)}
    dnums = lax.GatherDimensionNumbers(
        offset_dims=(), collapsed_slice_dims=(0,), start_index_map=(0,))

    def perm(x, k):
        return lax.gather(
            x, perms[k][:, None], dimension_numbers=dnums, slice_sizes=(1,),
            mode=lax.GatherScatterMode.PROMISE_IN_BOUNDS)

    def lane_sum(x):
        for k in (8, 4, 2, 1):
            x = x + perm(x, k)
        return x

    def row_dot(buf, row):
        prods = [u_rows[buf, row, pl.ds(fs * L, L)]
                 * r_rows[buf, row, pl.ds(fs * L, L)]
                 for fs in range(F // L)]
        while len(prods) > 1:
            prods = [a + b for a, b in zip(prods[::2], prods[1::2])]
        return lane_sum(prods[0])

    for c in range(NCH):
        buf = c % 2
        cu, cr = pending
        cu.wait()
        cr.wait()
        if c + 1 < NCH:
            pending = start(c + 1)

        def pair_body(j, outv, buf=buf, c=c):
            row = j * 2
            s0 = row_dot(buf, row)
            s1 = row_dot(buf, row + 1)
            l0 = row % L
            outv = jnp.where(lanes == l0, s0, outv)
            outv = jnp.where(lanes == l0 + 1, s1, outv)

            @pl.when(j % (L // 2) == (L // 2) - 1)
            def _store():
                out_v[pl.ds(c * CH + (j // (L // 2)) * L, L)] = outv
            return outv

        lax.fori_loop(0, CH // 2, pair_body, jnp.zeros((L,), jnp.float32))

    pltpu.sync_copy(out_v, out_hbm.at[pl.ds(base, BPW)])


@jax.jit
def _run(user, rsid, user_table, rsid_table):
    mesh = plsc.VectorSubcoreMesh(core_axis_name="c", subcore_axis_name="s")
    k = functools.partial(
        pl.kernel,
        out_type=jax.ShapeDtypeStruct((B,), jnp.float32),
        mesh=mesh,
        scratch_types=[
            pltpu.VMEM((NCH, CH), jnp.int32),
            pltpu.VMEM((NCH, CH), jnp.int32),
            pltpu.VMEM((2, CH, F), jnp.float32),
            pltpu.VMEM((2, CH, F), jnp.float32),
            pltpu.VMEM((BPW,), jnp.float32),
            pltpu.SemaphoreType.DMA,
            pltpu.SemaphoreType.DMA,
            pltpu.SemaphoreType.DMA,
        ],
    )(_body)
    return k(user, rsid, user_table, rsid_table)


def kernel(user, rsid, user_table, rsid_table):
    return _run(user, rsid, user_table, rsid_table)


# speedup vs baseline: 1.5491x; 1.0228x over previous
import functools

import jax
import jax.numpy as jnp
from jax import lax
from jax.experimental import pallas as pl
from jax.experimental.pallas import tpu as pltpu
from jax.experimental.pallas import tpu_sc as plsc

F = 128
B = 16384
NW = 32
BPW = B // NW
CH = 128
NCH = BPW // CH
L = 16
RG = 4


def _body(user_hbm, rsid_hbm, ut_hbm, rt_hbm, out_hbm,
          u_idx, r_idx, u_rows, r_rows, out_v, sem_u, sem_r, sem_i):
    wid = lax.axis_index("s") * 2 + lax.axis_index("c")
    base = wid * BPW

    pltpu.async_copy(user_hbm.at[pl.ds(base, CH)], u_idx.at[0], sem_u)
    pltpu.async_copy(rsid_hbm.at[pl.ds(base, CH)], r_idx.at[0], sem_r)
    rest = []
    for c in range(1, NCH):
        rest.append(pltpu.async_copy(
            user_hbm.at[pl.ds(base + c * CH, CH)], u_idx.at[c], sem_i))
        rest.append(pltpu.async_copy(
            rsid_hbm.at[pl.ds(base + c * CH, CH)], r_idx.at[c], sem_i))
    pltpu.make_async_copy(user_hbm.at[pl.ds(base, CH)], u_idx.at[0],
                          sem_u).wait()
    pltpu.make_async_copy(rsid_hbm.at[pl.ds(base, CH)], r_idx.at[0],
                          sem_r).wait()

    def start(c):
        buf = c % 2
        cu = pltpu.async_copy(ut_hbm.at[u_idx.at[c]], u_rows.at[buf], sem_u)
        cr = pltpu.async_copy(rt_hbm.at[r_idx.at[c]], r_rows.at[buf], sem_r)
        return cu, cr

    pending = start(0)
    for h in rest:
        h.wait()

    lanes = lax.iota(jnp.int32, L)
    perms = {k: lanes ^ k for k in (8, 4, 2, 1)}
    dnums = lax.GatherDimensionNumbers(
        offset_dims=(), collapsed_slice_dims=(0,), start_index_map=(0,))

    def perm(x, k):
        return lax.gather(
            x, perms[k][:, None], dimension_numbers=dnums, slice_sizes=(1,),
            mode=lax.GatherScatterMode.PROMISE_IN_BOUNDS)

    def lane_sum(x):
        for k in (8, 4, 2, 1):
            x = x + perm(x, k)
        return x

    def row_dot(buf, row):
        prods = [u_rows[buf, row, pl.ds(fs * L, L)]
                 * r_rows[buf, row, pl.ds(fs * L, L)]
                 for fs in range(F // L)]
        while len(prods) > 1:
            prods = [a + b for a, b in zip(prods[::2], prods[1::2])]
        return lane_sum(prods[0])

    for c in range(NCH):
        buf = c % 2
        cu, cr = pending
        cu.wait()
        cr.wait()
        if c + 1 < NCH:
            pending = start(c + 1)

        def group_body(j, outv, buf=buf, c=c):
            row0 = j * RG
            for i in range(RG):
                s = row_dot(buf, row0 + i)
                outv = jnp.where(lanes == (row0 % L) + i, s, outv)

            @pl.when(j % (L // RG) == (L // RG) - 1)
            def _store():
                out_v[pl.ds(c * CH + (j // (L // RG)) * L, L)] = outv
            return outv

        lax.fori_loop(0, CH // RG, group_body, jnp.zeros((L,), jnp.float32))

    pltpu.sync_copy(out_v, out_hbm.at[pl.ds(base, BPW)])


@jax.jit
def _run(user, rsid, user_table, rsid_table):
    mesh = plsc.VectorSubcoreMesh(core_axis_name="c", subcore_axis_name="s")
    k = functools.partial(
        pl.kernel,
        out_type=jax.ShapeDtypeStruct((B,), jnp.float32),
        mesh=mesh,
        scratch_types=[
            pltpu.VMEM((NCH, CH), jnp.int32),
            pltpu.VMEM((NCH, CH), jnp.int32),
            pltpu.VMEM((2, CH, F), jnp.float32),
            pltpu.VMEM((2, CH, F), jnp.float32),
            pltpu.VMEM((BPW,), jnp.float32),
            pltpu.SemaphoreType.DMA,
            pltpu.SemaphoreType.DMA,
            pltpu.SemaphoreType.DMA,
        ],
    )(_body)
    return k(user, rsid, user_table, rsid_table)


def kernel(user, rsid, user_table, rsid_table):
    return _run(user, rsid, user_table, rsid_table)
